# baseline (speedup 1.0000x reference)
import functools

import jax
import jax.numpy as jnp
from jax import lax
from jax.experimental import pallas as pl
from jax.experimental.pallas import tpu as pltpu
from jax.experimental.pallas import tpu_sc as plsc

_K = 1024
_D = 128
_BETA = 0.25
_TB = 1024

_NC = 2
_NS = 16
_NW = _NC * _NS


def _vq_block_kernel(z_ref, cb_ref, dist_ref, codes_ref, acc_ref, c2_ref):
    @pl.when(pl.program_id(0) == 0)
    def _init():
        cbf = cb_ref[...]
        c2_ref[...] = jnp.sum(cbf * cbf, axis=1).reshape(1, _K)
        acc_ref[...] = jnp.zeros_like(acc_ref)

    zb = z_ref[0]
    cb = cb_ref[...]

    cross = jax.lax.dot_general(
        zb, cb, (((0,), (1,)), ((), ())),
        preferred_element_type=jnp.float32,
        precision=jax.lax.Precision.DEFAULT)
    z2 = jnp.sum(zb * zb, axis=0)
    dist = z2[:, None] - 2.0 * cross + c2_ref[...]
    dist_ref[...] = dist

    min_d = jnp.min(dist, axis=1, keepdims=True)
    iota = jax.lax.broadcasted_iota(jnp.int32, dist.shape, 1)
    codes = jnp.min(jnp.where(dist == min_d, iota, _K), axis=1)
    codes_ref[0, 0] = codes

    acc_ref[...] += jnp.sum(min_d).reshape(1, 1)


def _sc_gather(codebook, idx_flat, n_rows):
    b_per_w = n_rows // _NW
    mesh = plsc.VectorSubcoreMesh(core_axis_name="c", subcore_axis_name="s")

    @functools.partial(
        pl.kernel, mesh=mesh,
        out_type=jax.ShapeDtypeStruct((n_rows, _D), jnp.float32),
        scratch_types=[
            pltpu.VMEM((b_per_w,), jnp.int32),
            pltpu.VMEM((b_per_w, _D), jnp.float32),
            pltpu.SemaphoreType.DMA,
        ],
    )
    def k(table_hbm, idx_hbm, out_hbm, idx_v, rows_v, sem):
        wid = lax.axis_index("s") * _NC + lax.axis_index("c")
        base = wid * b_per_w
        pltpu.sync_copy(idx_hbm.at[pl.ds(base, b_per_w)], idx_v)
        pltpu.async_copy(table_hbm.at[idx_v], rows_v, sem).wait()
        pltpu.sync_copy(rows_v, out_hbm.at[pl.ds(base, b_per_w)])

    return k(codebook, idx_flat)


def kernel(z, codebook):
    B, D, T = z.shape
    K = codebook.shape[0]
    n_blocks = (B * T) // _TB
    t_per_b = T // _TB

    grid = (n_blocks,)
    dist_flat, codes_blk, acc = pl.pallas_call(
        _vq_block_kernel,
        grid=grid,
        in_specs=[
            pl.BlockSpec((1, D, _TB), lambda i: (i // t_per_b, 0, i % t_per_b)),
            pl.BlockSpec((K, D), lambda i: (0, 0)),
        ],
        out_specs=[
            pl.BlockSpec((_TB, K), lambda i: (i, 0)),
            pl.BlockSpec((1, 1, _TB), lambda i: (i, 0, 0)),
            pl.BlockSpec((1, 1), lambda i: (0, 0)),
        ],
        out_shape=[
            jax.ShapeDtypeStruct((B * T, K), jnp.float32),
            jax.ShapeDtypeStruct((n_blocks, 1, _TB), jnp.int32),
            jax.ShapeDtypeStruct((1, 1), jnp.float32),
        ],
        scratch_shapes=[
            pltpu.VMEM((1, K), jnp.float32),
        ],
        compiler_params=pltpu.CompilerParams(
            dimension_semantics=("arbitrary",)),
    )(z, codebook)

    codes = codes_blk.reshape(B, T)
    zq_rows = _sc_gather(codebook, codes_blk.reshape(B * T), B * T)
    zqt = jnp.transpose(zq_rows.reshape(B, T, D), (0, 2, 1))
    loss = acc[0, 0] * (1.0 + _BETA) / (B * T * D)
    dist = dist_flat.reshape(B, T, K)
    return (zqt, codes, loss, dist)


# speedup vs baseline: 1.7044x; 1.7044x over previous
import jax
import jax.numpy as jnp
from jax.experimental import pallas as pl
from jax.experimental.pallas import tpu as pltpu

_K = 1024
_D = 128
_BETA = 0.25
_TB = 1024


def _vq_block_kernel(z_ref, cb_ref, dist_ref, codes_ref, zqt_ref, acc_ref,
                     c2_ref, cbh_ref):
    @pl.when(pl.program_id(0) == 0)
    def _init():
        cbf = cb_ref[...]
        c2_ref[...] = jnp.sum(cbf * cbf, axis=1).reshape(1, _K)
        cbh_ref[...] = cbf.astype(jnp.bfloat16)
        acc_ref[...] = jnp.zeros_like(acc_ref)

    zb = z_ref[0]
    cb = cb_ref[...]

    cross = jax.lax.dot_general(
        zb, cb, (((0,), (1,)), ((), ())),
        preferred_element_type=jnp.float32,
        precision=jax.lax.Precision.DEFAULT)
    z2 = jnp.sum(zb * zb, axis=0)
    dist = z2[:, None] - 2.0 * cross + c2_ref[...]
    dist_ref[...] = dist

    min_d = jnp.min(dist, axis=1, keepdims=True)
    iota = jax.lax.broadcasted_iota(jnp.int32, dist.shape, 1)
    codes = jnp.min(jnp.where(dist == min_d, iota, _K), axis=1)
    codes_ref[0, 0] = codes

    onehot = (iota == codes[:, None]).astype(jnp.bfloat16)
    zqt_ref[0] = jax.lax.dot_general(
        cbh_ref[...], onehot, (((0,), (1,)), ((), ())),
        preferred_element_type=jnp.float32,
        precision=jax.lax.Precision.DEFAULT)

    acc_ref[...] += jnp.sum(min_d).reshape(1, 1)


def kernel(z, codebook):
    B, D, T = z.shape
    K = codebook.shape[0]
    n_blocks = (B * T) // _TB
    t_per_b = T // _TB

    grid = (n_blocks,)
    dist_flat, codes_blk, zqt, acc = pl.pallas_call(
        _vq_block_kernel,
        grid=grid,
        in_specs=[
            pl.BlockSpec((1, D, _TB), lambda i: (i // t_per_b, 0, i % t_per_b)),
            pl.BlockSpec((K, D), lambda i: (0, 0)),
        ],
        out_specs=[
            pl.BlockSpec((_TB, K), lambda i: (i, 0)),
            pl.BlockSpec((1, 1, _TB), lambda i: (i, 0, 0)),
            pl.BlockSpec((1, D, _TB), lambda i: (i // t_per_b, 0, i % t_per_b)),
            pl.BlockSpec((1, 1), lambda i: (0, 0)),
        ],
        out_shape=[
            jax.ShapeDtypeStruct((B * T, K), jnp.float32),
            jax.ShapeDtypeStruct((n_blocks, 1, _TB), jnp.int32),
            jax.ShapeDtypeStruct((B, D, T), jnp.float32),
            jax.ShapeDtypeStruct((1, 1), jnp.float32),
        ],
        scratch_shapes=[
            pltpu.VMEM((1, K), jnp.float32),
            pltpu.VMEM((K, D), jnp.bfloat16),
        ],
        compiler_params=pltpu.CompilerParams(
            dimension_semantics=("arbitrary",)),
    )(z, codebook)

    codes = codes_blk.reshape(B, T)
    loss = acc[0, 0] * (1.0 + _BETA) / (B * T * D)
    dist = dist_flat.reshape(B, T, K)
    return (zqt, codes, loss, dist)


# speedup vs baseline: 1.7447x; 1.0236x over previous
import jax
import jax.numpy as jnp
from jax.experimental import pallas as pl
from jax.experimental.pallas import tpu as pltpu

_K = 1024
_D = 128
_BETA = 0.25
_TB = 1024


def _vq_block_kernel(z_ref, cb_ref, dist_ref, codes_ref, zqt_ref, acc_ref,
                     c2_ref, cbh_ref, iota_ref):
    @pl.when(pl.program_id(0) == 0)
    def _init():
        cbf = cb_ref[...]
        c2_ref[...] = jnp.sum(cbf * cbf, axis=1).reshape(1, _K)
        cbh_ref[...] = cbf.astype(jnp.bfloat16)
        acc_ref[...] = jnp.zeros_like(acc_ref)
        iota_ref[...] = jax.lax.broadcasted_iota(
            jnp.int32, (_TB, _K), 1).astype(jnp.float32)

    zb = z_ref[0]
    cb = cb_ref[...]

    cross = jax.lax.dot_general(
        zb, cb, (((0,), (1,)), ((), ())),
        preferred_element_type=jnp.float32,
        precision=jax.lax.Precision.DEFAULT)
    z2 = jnp.sum(zb * zb, axis=0)
    dist = z2[:, None] - 2.0 * cross + c2_ref[...]
    dist_ref[...] = dist

    min_d = jnp.min(dist, axis=1, keepdims=True)
    iota = iota_ref[...]
    codes_f = jnp.min(jnp.where(dist == min_d, iota, jnp.float32(_K)),
                      axis=1)
    codes_ref[0, 0] = codes_f.astype(jnp.int32)

    onehot = (iota == codes_f[:, None]).astype(jnp.bfloat16)
    zqt_ref[0] = jax.lax.dot_general(
        cbh_ref[...], onehot, (((0,), (1,)), ((), ())),
        preferred_element_type=jnp.float32,
        precision=jax.lax.Precision.DEFAULT)

    acc_ref[...] += jnp.sum(min_d).reshape(1, 1)


def kernel(z, codebook):
    B, D, T = z.shape
    K = codebook.shape[0]
    n_blocks = (B * T) // _TB
    t_per_b = T // _TB

    grid = (n_blocks,)
    dist_flat, codes_blk, zqt, acc = pl.pallas_call(
        _vq_block_kernel,
        grid=grid,
        in_specs=[
            pl.BlockSpec((1, D, _TB), lambda i: (i // t_per_b, 0, i % t_per_b)),
            pl.BlockSpec((K, D), lambda i: (0, 0)),
        ],
        out_specs=[
            pl.BlockSpec((_TB, K), lambda i: (i, 0)),
            pl.BlockSpec((1, 1, _TB), lambda i: (i, 0, 0)),
            pl.BlockSpec((1, D, _TB), lambda i: (i // t_per_b, 0, i % t_per_b)),
            pl.BlockSpec((1, 1), lambda i: (0, 0)),
        ],
        out_shape=[
            jax.ShapeDtypeStruct((B * T, K), jnp.float32),
            jax.ShapeDtypeStruct((n_blocks, 1, _TB), jnp.int32),
            jax.ShapeDtypeStruct((B, D, T), jnp.float32),
            jax.ShapeDtypeStruct((1, 1), jnp.float32),
        ],
        scratch_shapes=[
            pltpu.VMEM((1, K), jnp.float32),
            pltpu.VMEM((K, D), jnp.bfloat16),
            pltpu.VMEM((_TB, K), jnp.float32),
        ],
        compiler_params=pltpu.CompilerParams(
            dimension_semantics=("arbitrary",)),
    )(z, codebook)

    codes = codes_blk.reshape(B, T)
    loss = acc[0, 0] * (1.0 + _BETA) / (B * T * D)
    dist = dist_flat.reshape(B, T, K)
    return (zqt, codes, loss, dist)
